# baseline (speedup 1.0000x reference)
import functools

import jax
import jax.numpy as jnp
from jax.experimental import pallas as pl


def _assign_block(b_ref, g_ref, t_ref, bbox_ref, lab_ref, *, m: int):
    b = b_ref[...]
    g = g_ref[...]
    table = t_ref[...]

    bx1, by1, bx2, by2 = b[:, 0:1], b[:, 1:2], b[:, 2:3], b[:, 3:4]
    gx1, gy1, gx2, gy2 = g[0:1, :], g[1:2, :], g[2:3, :], g[3:4, :]

    w = jnp.maximum(jnp.minimum(bx2, gx2) - jnp.maximum(bx1, gx1), 0.0)
    h = jnp.maximum(jnp.minimum(by2, gy2) - jnp.maximum(by1, gy1), 0.0)
    inter = w * h
    area_b = (bx2 - bx1) * (by2 - by1)
    area_g = (gx2 - gx1) * (gy2 - gy1)
    union = jnp.maximum(area_b + area_g - inter, 1e-7)
    lane = jax.lax.broadcasted_iota(jnp.int32, inter.shape, 1)
    code = jnp.where(inter >= 0.5 * union, lane,
                     jnp.where(inter < 0.3 * union, 3 * m, m))
    r = jnp.min(code, axis=1, keepdims=True)
    pos_any = r < m
    neg_all = r >= 3 * m

    onehot = (lane == r).astype(jnp.float32)
    sel = jnp.dot(onehot, table, preferred_element_type=jnp.float32,
                  precision=jax.lax.Precision.HIGHEST)

    neg_one = jnp.float32(-1.0)
    bbox_ref[...] = jnp.where(pos_any, sel[:, 0:4], neg_one)
    labf = jnp.where(pos_any, jnp.round(sel[:, 4:5]),
                     jnp.where(neg_all, 0.0, neg_one))
    lab_ref[...] = labf.astype(jnp.int32)


def kernel(bboxes, gt_bboxes, gt_labels):
    n = bboxes.shape[0]
    m = gt_bboxes.shape[0]
    tile = 400
    grid = (n + tile - 1) // tile

    gt_t = gt_bboxes.T
    labf = gt_labels.astype(jnp.float32)
    table = jnp.concatenate(
        [gt_bboxes, labf[:, None], jnp.zeros((m, 3), jnp.float32)], axis=1)

    bbox_out, lab_out = pl.pallas_call(
        functools.partial(_assign_block, m=m),
        grid=(grid,),
        in_specs=[
            pl.BlockSpec((tile, 4), lambda i: (i, 0)),
            pl.BlockSpec((4, m), lambda i: (0, 0)),
            pl.BlockSpec((m, 8), lambda i: (0, 0)),
        ],
        out_specs=[
            pl.BlockSpec((tile, 4), lambda i: (i, 0)),
            pl.BlockSpec((tile, 1), lambda i: (i, 0)),
        ],
        out_shape=[
            jax.ShapeDtypeStruct((n, 4), jnp.float32),
            jax.ShapeDtypeStruct((n, 1), jnp.int32),
        ],
    )(bboxes, gt_t, table)

    return lab_out.reshape(n), bbox_out


# speedup vs baseline: 1.2988x; 1.2988x over previous
import functools

import jax
import jax.numpy as jnp
from jax.experimental import pallas as pl


def _assign_block(b_ref, g_ref, t_ref, bbox_ref, lab_ref, *, m: int):
    b = b_ref[...]
    g = g_ref[...]
    table = t_ref[...]

    bx1, by1, bx2, by2 = b[:, 0:1], b[:, 1:2], b[:, 2:3], b[:, 3:4]
    gx1, gy1, gx2, gy2 = g[0:1, :], g[1:2, :], g[2:3, :], g[3:4, :]

    w = jnp.maximum(jnp.minimum(bx2, gx2) - jnp.maximum(bx1, gx1), 0.0)
    h = jnp.maximum(jnp.minimum(by2, gy2) - jnp.maximum(by1, gy1), 0.0)
    inter = w * h
    area_b = (bx2 - bx1) * (by2 - by1)
    area_g = (gx2 - gx1) * (gy2 - gy1)
    union = jnp.maximum(area_b + area_g - inter, 1e-7)
    lane = jax.lax.broadcasted_iota(jnp.int32, inter.shape, 1)
    code = jnp.where(inter >= 0.5 * union, lane,
                     jnp.where(inter < 0.3 * union, 3 * m, m))
    r = jnp.min(code, axis=1, keepdims=True)
    pos_any = r < m
    neg_all = r >= 3 * m

    onehot = (lane == r).astype(jnp.float32)
    sel = jnp.dot(onehot, table, preferred_element_type=jnp.float32)

    neg_one = jnp.float32(-1.0)
    bbox_ref[...] = jnp.where(pos_any, sel[:, 0:4], neg_one)
    labf = jnp.where(pos_any, jnp.round(sel[:, 4:5]),
                     jnp.where(neg_all, 0.0, neg_one))
    lab_ref[...] = labf.astype(jnp.int32)


def kernel(bboxes, gt_bboxes, gt_labels):
    n = bboxes.shape[0]
    m = gt_bboxes.shape[0]
    tile = 400
    grid = (n + tile - 1) // tile

    gt_t = gt_bboxes.T
    labf = gt_labels.astype(jnp.float32)
    table = jnp.concatenate(
        [gt_bboxes, labf[:, None], jnp.zeros((m, 3), jnp.float32)], axis=1)

    bbox_out, lab_out = pl.pallas_call(
        functools.partial(_assign_block, m=m),
        grid=(grid,),
        in_specs=[
            pl.BlockSpec((tile, 4), lambda i: (i, 0)),
            pl.BlockSpec((4, m), lambda i: (0, 0)),
            pl.BlockSpec((m, 8), lambda i: (0, 0)),
        ],
        out_specs=[
            pl.BlockSpec((tile, 4), lambda i: (i, 0)),
            pl.BlockSpec((tile, 1), lambda i: (i, 0)),
        ],
        out_shape=[
            jax.ShapeDtypeStruct((n, 4), jnp.float32),
            jax.ShapeDtypeStruct((n, 1), jnp.int32),
        ],
    )(bboxes, gt_t, table)

    return lab_out.reshape(n), bbox_out


# speedup vs baseline: 1.6955x; 1.3055x over previous
import functools

import jax
import jax.numpy as jnp
from jax.experimental import pallas as pl


def _assign_block(b_ref, g_ref, t_ref, bbox_ref, lab_ref, *, m: int):
    b = b_ref[...]
    g = g_ref[...]
    table = t_ref[...]

    bx1, by1, bx2, by2 = b[:, 0:1], b[:, 1:2], b[:, 2:3], b[:, 3:4]
    gx1, gy1, gx2, gy2 = g[0:1, :], g[1:2, :], g[2:3, :], g[3:4, :]

    w = jnp.maximum(jnp.minimum(bx2, gx2) - jnp.maximum(bx1, gx1), 0.0)
    h = jnp.maximum(jnp.minimum(by2, gy2) - jnp.maximum(by1, gy1), 0.0)
    inter = w * h
    area_b = (bx2 - bx1) * (by2 - by1)
    area_g = (gx2 - gx1) * (gy2 - gy1)
    union = jnp.maximum(area_b + area_g - inter, 1e-7)
    lane = jax.lax.broadcasted_iota(jnp.int32, inter.shape, 1)
    code = jnp.where(inter >= 0.5 * union, lane,
                     jnp.where(inter < 0.3 * union, 3 * m, m))
    r = jnp.min(code, axis=1, keepdims=True)
    pos_any = r < m
    neg_all = r >= 3 * m

    onehot = (lane == r).astype(jnp.float32)
    sel = jnp.dot(onehot, table, preferred_element_type=jnp.float32)

    neg_one = jnp.float32(-1.0)
    bbox_ref[...] = jnp.where(pos_any, sel[:, 0:4], neg_one)
    labf = jnp.where(pos_any, jnp.round(sel[:, 4:5]),
                     jnp.where(neg_all, 0.0, neg_one))
    lab_ref[...] = labf.astype(jnp.int32)


def kernel(bboxes, gt_bboxes, gt_labels):
    n = bboxes.shape[0]
    m = gt_bboxes.shape[0]
    tile = 2000
    grid = (n + tile - 1) // tile

    gt_t = gt_bboxes.T
    labf = gt_labels.astype(jnp.float32)
    table = jnp.concatenate(
        [gt_bboxes, labf[:, None], jnp.zeros((m, 3), jnp.float32)], axis=1)

    bbox_out, lab_out = pl.pallas_call(
        functools.partial(_assign_block, m=m),
        grid=(grid,),
        in_specs=[
            pl.BlockSpec((tile, 4), lambda i: (i, 0)),
            pl.BlockSpec((4, m), lambda i: (0, 0)),
            pl.BlockSpec((m, 8), lambda i: (0, 0)),
        ],
        out_specs=[
            pl.BlockSpec((tile, 4), lambda i: (i, 0)),
            pl.BlockSpec((tile, 1), lambda i: (i, 0)),
        ],
        out_shape=[
            jax.ShapeDtypeStruct((n, 4), jnp.float32),
            jax.ShapeDtypeStruct((n, 1), jnp.int32),
        ],
    )(bboxes, gt_t, table)

    return lab_out.reshape(n), bbox_out
